# baseline (speedup 1.0000x reference)
import jax
import jax.numpy as jnp
from jax.experimental import pallas as pl
from jax.experimental.pallas import tpu as pltpu


def _l2norm(x, eps=1e-12):
    d = jnp.sqrt(jnp.sum(x * x, axis=(2, 3), keepdims=True))
    return x / jnp.maximum(d, eps)


def _stats(x):
    rgb, ir = x[0], x[1]
    rgb_cap = jnp.mean(rgb, axis=1, keepdims=True)
    rgb_cmp = jnp.max(rgb, axis=1, keepdims=True)
    ir_cap = jnp.mean(ir, axis=1, keepdims=True)
    ir_cmp = jnp.max(ir, axis=1, keepdims=True)
    x1_cp = jnp.concatenate([rgb_cap, rgb_cmp], axis=1)
    x2_cp = jnp.concatenate([ir_cap, ir_cmp], axis=1)
    cp = x1_cp + x2_cp
    sa = jnp.maximum(cp[:, ::2, :, :], cp[:, 1::2, :, :])
    sa_sig = jax.nn.sigmoid(sa)
    sa_norm = _l2norm(sa_sig)
    sim_rgb = jnp.sum(sa_norm * _l2norm(rgb), axis=(2, 3))
    sim_ir = jnp.sum(sa_norm * _l2norm(ir), axis=(2, 3))
    return sa, sim_rgb, sim_ir


def _kmain(srow_ref, scol_ref, sims_ref, x_ref, sig_ref, out0_ref, out1_ref,
           minrow_ref):
    C = x_ref.shape[2]
    s = pl.program_id(1)
    srow = srow_ref[0, 0]
    scol = scol_ref[0, 0]
    iota_row = jax.lax.broadcasted_iota(jnp.int32, (1, C), 1)
    iota_col = jax.lax.broadcasted_iota(jnp.int32, (C, 1), 0)
    before = (scol < srow) | ((scol == srow) & (iota_col < iota_row))
    rank = jnp.sum(jnp.where(before, 1.0, 0.0), axis=0,
                   keepdims=True).astype(jnp.int32)

    allsims = sims_ref[...]
    cnt = jnp.sum(jnp.where(allsims > 0, 1.0, 0.0), axis=3)
    k0 = jnp.max(cnt[0]).astype(jnp.int32)
    k1 = jnp.max(cnt[1]).astype(jnp.int32)
    act0 = (k1 > k0) & (k0 > 0)
    act1 = (k0 > k1) & (k1 > 0)
    is0 = s == 0
    act = jnp.where(is0, act0, act1)
    kk = jnp.where(is0, k0, k1)

    pos_act = jnp.where(rank < kk, rank,
                        jnp.where(rank == C - 1, kk, rank + 1))
    pos = jnp.where(act, pos_act, rank)

    xb = x_ref[0, 0]
    sig = sig_ref[0, 0]
    P = (iota_col == pos).astype(jnp.float32)
    out = jax.lax.dot_general(
        P, xb, (((1,), (0,)), ((), ())),
        precision=jax.lax.Precision.HIGHEST,
        preferred_element_type=jnp.float32) * sig

    ohmin = (rank == 0).astype(jnp.float32)
    rowmin = jax.lax.dot_general(
        ohmin, xb, (((1,), (0,)), ((), ())),
        preferred_element_type=jnp.float32)

    @pl.when(is0)
    def _():
        out0_ref[0] = out
        minrow_ref[...] = rowmin

    @pl.when(jnp.logical_not(is0))
    def _():
        ef = jnp.maximum(minrow_ref[...], rowmin)
        out1_ref[0] = jnp.where(act1 & (iota_col == k1), ef * sig, out)

        @pl.when(act0)
        def _():
            out0_ref[0] = jnp.where(iota_col == k0, ef * sig, out0_ref[0])


def kernel(x):
    S, B, C, H, W = x.shape
    HW = H * W
    f32 = jnp.float32

    sa, sim_rgb, sim_ir = _stats(x)
    sa_sig = jax.nn.sigmoid(sa)
    sims = jnp.stack([sim_rgb, sim_ir]).reshape(S, B, 1, C)
    sims_col = sims.reshape(S, B, C, 1)
    sig_arr = sa_sig.reshape(B, 1, HW)
    xr = x.reshape(S, B, C, HW)

    out0, out1 = pl.pallas_call(
        _kmain,
        grid=(B, S),
        in_specs=[
            pl.BlockSpec((1, 1, 1, C), lambda b, s: (s, b, 0, 0)),
            pl.BlockSpec((1, 1, C, 1), lambda b, s: (s, b, 0, 0)),
            pl.BlockSpec((S, B, 1, C), lambda b, s: (0, 0, 0, 0)),
            pl.BlockSpec((1, 1, C, HW), lambda b, s: (s, b, 0, 0)),
            pl.BlockSpec((1, 1, HW), lambda b, s: (b, 0, 0)),
        ],
        out_specs=[pl.BlockSpec((1, C, HW), lambda b, s: (b, 0, 0)),
                   pl.BlockSpec((1, C, HW), lambda b, s: (b, 0, 0))],
        out_shape=[jax.ShapeDtypeStruct((B, C, HW), f32),
                   jax.ShapeDtypeStruct((B, C, HW), f32)],
        scratch_shapes=[pltpu.VMEM((1, HW), f32)],
    )(sims, sims_col, sims, xr, sig_arr)

    return out0.reshape(B, C, H, W), out1.reshape(B, C, H, W)


# speedup vs baseline: 1.4246x; 1.4246x over previous
import jax
import jax.numpy as jnp
from jax.experimental import pallas as pl
from jax.experimental.pallas import tpu as pltpu


def _l2norm(x, eps=1e-12):
    d = jnp.sqrt(jnp.sum(x * x, axis=(2, 3), keepdims=True))
    return x / jnp.maximum(d, eps)


def _stats(x):
    rgb, ir = x[0], x[1]
    rgb_cap = jnp.mean(rgb, axis=1, keepdims=True)
    rgb_cmp = jnp.max(rgb, axis=1, keepdims=True)
    ir_cap = jnp.mean(ir, axis=1, keepdims=True)
    ir_cmp = jnp.max(ir, axis=1, keepdims=True)
    x1_cp = jnp.concatenate([rgb_cap, rgb_cmp], axis=1)
    x2_cp = jnp.concatenate([ir_cap, ir_cmp], axis=1)
    cp = x1_cp + x2_cp
    sa = jnp.maximum(cp[:, ::2, :, :], cp[:, 1::2, :, :])
    sa_sig = jax.nn.sigmoid(sa)
    sa_norm = _l2norm(sa_sig)
    sim_rgb = jnp.sum(sa_norm * _l2norm(rgb), axis=(2, 3))
    sim_ir = jnp.sum(sa_norm * _l2norm(ir), axis=(2, 3))
    return sa, sim_rgb, sim_ir


def _kmain(srow_ref, scol_ref, sims_ref, x_ref, sig_ref, out0_ref, out1_ref,
           minrow_ref):
    C = x_ref.shape[2]
    s = pl.program_id(1)
    srow = srow_ref[0, 0]
    scol = scol_ref[0, 0]
    iota_row = jax.lax.broadcasted_iota(jnp.int32, (1, C), 1)
    iota_col = jax.lax.broadcasted_iota(jnp.int32, (C, 1), 0)
    before = (scol < srow) | ((scol == srow) & (iota_col < iota_row))
    rank = jnp.sum(jnp.where(before, 1.0, 0.0), axis=0,
                   keepdims=True).astype(jnp.int32)

    allsims = sims_ref[...]
    cnt = jnp.sum(jnp.where(allsims > 0, 1.0, 0.0), axis=3)
    k0 = jnp.max(cnt[0]).astype(jnp.int32)
    k1 = jnp.max(cnt[1]).astype(jnp.int32)
    act0 = (k1 > k0) & (k0 > 0)
    act1 = (k0 > k1) & (k1 > 0)
    is0 = s == 0
    act = jnp.where(is0, act0, act1)
    kk = jnp.where(is0, k0, k1)

    pos_act = jnp.where(rank < kk, rank,
                        jnp.where(rank == C - 1, kk, rank + 1))
    pos = jnp.where(act, pos_act, rank)

    xb = x_ref[0, 0]
    sig = sig_ref[0, 0]
    P = (iota_col == pos).astype(jnp.float32)
    out = jax.lax.dot_general(
        P, xb, (((1,), (0,)), ((), ())),
        preferred_element_type=jnp.float32) * sig

    ohmin = (rank == 0).astype(jnp.float32)
    rowmin = jax.lax.dot_general(
        ohmin, xb, (((1,), (0,)), ((), ())),
        preferred_element_type=jnp.float32)

    @pl.when(is0)
    def _():
        out0_ref[0] = out
        minrow_ref[...] = rowmin

    @pl.when(jnp.logical_not(is0))
    def _():
        ef = jnp.maximum(minrow_ref[...], rowmin)
        out1_ref[0] = jnp.where(act1 & (iota_col == k1), ef * sig, out)

        @pl.when(act0)
        def _():
            out0_ref[0] = jnp.where(iota_col == k0, ef * sig, out0_ref[0])


def kernel(x):
    S, B, C, H, W = x.shape
    HW = H * W
    f32 = jnp.float32

    sa, sim_rgb, sim_ir = _stats(x)
    sa_sig = jax.nn.sigmoid(sa)
    sims = jnp.stack([sim_rgb, sim_ir]).reshape(S, B, 1, C)
    sims_col = sims.reshape(S, B, C, 1)
    sig_arr = sa_sig.reshape(B, 1, HW)
    xr = x.reshape(S, B, C, HW)

    out0, out1 = pl.pallas_call(
        _kmain,
        grid=(B, S),
        in_specs=[
            pl.BlockSpec((1, 1, 1, C), lambda b, s: (s, b, 0, 0)),
            pl.BlockSpec((1, 1, C, 1), lambda b, s: (s, b, 0, 0)),
            pl.BlockSpec((S, B, 1, C), lambda b, s: (0, 0, 0, 0)),
            pl.BlockSpec((1, 1, C, HW), lambda b, s: (s, b, 0, 0)),
            pl.BlockSpec((1, 1, HW), lambda b, s: (b, 0, 0)),
        ],
        out_specs=[pl.BlockSpec((1, C, HW), lambda b, s: (b, 0, 0)),
                   pl.BlockSpec((1, C, HW), lambda b, s: (b, 0, 0))],
        out_shape=[jax.ShapeDtypeStruct((B, C, HW), f32),
                   jax.ShapeDtypeStruct((B, C, HW), f32)],
        scratch_shapes=[pltpu.VMEM((1, HW), f32)],
    )(sims, sims_col, sims, xr, sig_arr)

    return out0.reshape(B, C, H, W), out1.reshape(B, C, H, W)
